# baseline (speedup 1.0000x reference)
import functools

import numpy as np
import jax
import jax.numpy as jnp
from jax.experimental import pallas as pl
from jax.experimental.pallas import tpu as pltpu


def _threefry2x32(k1, k2, x0, x1):
    rot = (13, 15, 26, 6, 17, 29, 16, 24)
    ks0 = np.uint32(k1)
    ks1 = np.uint32(k2)
    ks2 = np.uint32(ks0 ^ ks1 ^ np.uint32(0x1BD11BDA))
    x0 = x0 + ks0
    x1 = x1 + ks1

    def rnd(a, b, r):
        a = a + b
        b = (b << np.uint32(r)) | (b >> np.uint32(32 - r))
        return a, a ^ b

    for r in rot[0:4]:
        x0, x1 = rnd(x0, x1, r)
    x0 = x0 + ks1; x1 = x1 + ks2 + np.uint32(1)
    for r in rot[4:8]:
        x0, x1 = rnd(x0, x1, r)
    x0 = x0 + ks2; x1 = x1 + ks0 + np.uint32(2)
    for r in rot[0:4]:
        x0, x1 = rnd(x0, x1, r)
    x0 = x0 + ks0; x1 = x1 + ks1 + np.uint32(3)
    for r in rot[4:8]:
        x0, x1 = rnd(x0, x1, r)
    x0 = x0 + ks1; x1 = x1 + ks2 + np.uint32(4)
    for r in rot[0:4]:
        x0, x1 = rnd(x0, x1, r)
    x0 = x0 + ks2; x1 = x1 + ks0 + np.uint32(5)
    return x0, x1


@functools.lru_cache(maxsize=2)
def _gumbel_uniform(seed, n):
    old = np.seterr(over="ignore")
    try:
        i = np.arange(n, dtype=np.uint32)
        o0, o1 = _threefry2x32(np.uint32(seed >> 32),
                               np.uint32(seed & 0xFFFFFFFF), np.uint32(0), i)
        bits = o0 ^ o1
        fb = (bits >> np.uint32(9)) | np.uint32(0x3F800000)
        u = fb.view(np.float32) - np.float32(1.0)
        return np.maximum(np.float32(1e-20),
                          u * np.float32(1.0 - 1e-20) + np.float32(1e-20))
    finally:
        np.seterr(**old)


def _assign_kernel(x_ref, c_ref, x2_ref, c2_ref, u_ref, o_ref):
    x = x_ref[...]
    cm = c_ref[...]
    dot = jax.lax.dot_general(
        x, cm, (((1,), (1,)), ((), ())),
        preferred_element_type=jnp.float32)
    x2 = x2_ref[...]
    c2 = c2_ref[...]
    d2 = jnp.maximum(x2 + c2 - 2.0 * dot, 1e-12)
    g = -jnp.log(-jnp.log(u_ref[...]))
    z = g - jnp.sqrt(d2)
    idx = jnp.argmax(z, axis=1)
    iota = jax.lax.broadcasted_iota(jnp.int32, z.shape, 1)
    o_ref[...] = (iota == idx[:, None]).astype(jnp.float32)


def kernel(pixel_features, cluster_centers):
    B, N, D = pixel_features.shape
    K = cluster_centers.shape[1]
    x = pixel_features.reshape(B * N, D)
    c = cluster_centers[0]
    u = _gumbel_uniform(12345, B * N * K).reshape(B * N, K)
    x2 = jnp.sum(x * x, axis=-1, keepdims=True)
    c2 = jnp.sum(c * c, axis=-1)[None, :]

    T = 1024
    out = pl.pallas_call(
        _assign_kernel,
        grid=(B * N // T,),
        in_specs=[
            pl.BlockSpec((T, D), lambda i: (i, 0)),
            pl.BlockSpec((K, D), lambda i: (0, 0)),
            pl.BlockSpec((T, 1), lambda i: (i, 0)),
            pl.BlockSpec((1, K), lambda i: (0, 0)),
            pl.BlockSpec((T, K), lambda i: (i, 0)),
        ],
        out_specs=pl.BlockSpec((T, K), lambda i: (i, 0)),
        out_shape=jax.ShapeDtypeStruct((B * N, K), jnp.float32),
        compiler_params=pltpu.CompilerParams(
            dimension_semantics=("parallel",)),
    )(x, c, x2, c2, u)
    return out.reshape(B, N, K)


# speedup vs baseline: 1.2321x; 1.2321x over previous
import functools

import numpy as np
import jax
import jax.numpy as jnp
from jax.experimental import pallas as pl
from jax.experimental.pallas import tpu as pltpu


def _threefry2x32(k1, k2, x0, x1):
    rot = (13, 15, 26, 6, 17, 29, 16, 24)
    ks0 = np.uint32(k1)
    ks1 = np.uint32(k2)
    ks2 = np.uint32(ks0 ^ ks1 ^ np.uint32(0x1BD11BDA))
    x0 = x0 + ks0
    x1 = x1 + ks1

    def rnd(a, b, r):
        a = a + b
        b = (b << np.uint32(r)) | (b >> np.uint32(32 - r))
        return a, a ^ b

    for r in rot[0:4]:
        x0, x1 = rnd(x0, x1, r)
    x0 = x0 + ks1; x1 = x1 + ks2 + np.uint32(1)
    for r in rot[4:8]:
        x0, x1 = rnd(x0, x1, r)
    x0 = x0 + ks2; x1 = x1 + ks0 + np.uint32(2)
    for r in rot[0:4]:
        x0, x1 = rnd(x0, x1, r)
    x0 = x0 + ks0; x1 = x1 + ks1 + np.uint32(3)
    for r in rot[4:8]:
        x0, x1 = rnd(x0, x1, r)
    x0 = x0 + ks1; x1 = x1 + ks2 + np.uint32(4)
    for r in rot[0:4]:
        x0, x1 = rnd(x0, x1, r)
    x0 = x0 + ks2; x1 = x1 + ks0 + np.uint32(5)
    return x0, x1


@functools.lru_cache(maxsize=2)
def _gumbel_uniform(seed, n):
    old = np.seterr(over="ignore")
    try:
        i = np.arange(n, dtype=np.uint32)
        o0, o1 = _threefry2x32(np.uint32(seed >> 32),
                               np.uint32(seed & 0xFFFFFFFF), np.uint32(0), i)
        bits = o0 ^ o1
        fb = (bits >> np.uint32(9)) | np.uint32(0x3F800000)
        u = fb.view(np.float32) - np.float32(1.0)
        return np.maximum(np.float32(1e-20),
                          u * np.float32(1.0 - 1e-20) + np.float32(1e-20))
    finally:
        np.seterr(**old)


def _assign_kernel(x_ref, c_ref, u_ref, o_ref):
    x = x_ref[...]
    cm = c_ref[...]
    dot = jax.lax.dot_general(
        x, cm, (((1,), (1,)), ((), ())),
        preferred_element_type=jnp.float32)
    x2 = jnp.sum(x * x, axis=1, keepdims=True)
    c2 = jnp.sum(cm * cm, axis=1)
    d2 = jnp.maximum(x2 + c2[None, :] - 2.0 * dot, 1e-12)
    g = -jnp.log(-jnp.log(u_ref[...]))
    z = g - jnp.sqrt(d2)
    idx = jnp.argmax(z, axis=1)
    iota = jax.lax.broadcasted_iota(jnp.int32, z.shape, 1)
    o_ref[...] = (iota == idx[:, None]).astype(jnp.float32)


def kernel(pixel_features, cluster_centers):
    B, N, D = pixel_features.shape
    K = cluster_centers.shape[1]
    x = pixel_features.reshape(B * N, D)
    c = cluster_centers[0]
    u = _gumbel_uniform(12345, B * N * K).reshape(B * N, K)

    T = 1024
    out = pl.pallas_call(
        _assign_kernel,
        grid=(B * N // T,),
        in_specs=[
            pl.BlockSpec((T, D), lambda i: (i, 0)),
            pl.BlockSpec((K, D), lambda i: (0, 0)),
            pl.BlockSpec((T, K), lambda i: (i, 0)),
        ],
        out_specs=pl.BlockSpec((T, K), lambda i: (i, 0)),
        out_shape=jax.ShapeDtypeStruct((B * N, K), jnp.float32),
        compiler_params=pltpu.CompilerParams(
            dimension_semantics=("parallel",)),
    )(x, c, u)
    return out.reshape(B, N, K)
